# baseline (speedup 1.0000x reference)
import functools

import jax
import jax.numpy as jnp
from jax import lax
from jax.experimental import pallas as pl
from jax.experimental.pallas import tpu as pltpu
from jax.experimental.pallas import tpu_sc as plsc

V = 100000
D = 128
B = 1024
W = 20

NC = 2
NS = 16
NW = NC * NS
BPW = B // NW
CR = BPW * W
CHUNK = 128
L = 16
NV = D // L

TV = 8192
NT = (V + TV - 1) // TV
LOG2E = 1.4426950408889634
NPAD = NT * TV - V


def _rsqrt_bcast(tot):
  i = lax.bitcast_convert_type(tot, jnp.int32)
  y0 = lax.bitcast_convert_type(
      jnp.int32(0x5F3759DF) - lax.shift_right_logical(i, 1), jnp.float32)
  y = jnp.full((L,), y0, jnp.float32)
  t = jnp.full((L,), tot, jnp.float32)
  for _ in range(2):
    y = y * (1.5 - 0.5 * t * y * y)
  return y


def _sc_gather_normalize(table, idx_ctx, idx_cen):
  mesh = plsc.VectorSubcoreMesh(core_axis_name="c", subcore_axis_name="s")

  @functools.partial(
      pl.kernel,
      out_type=(
          jax.ShapeDtypeStruct((B, D), jnp.float32),
          jax.ShapeDtypeStruct((B, D), jnp.float32),
      ),
      mesh=mesh,
      scratch_types=[
          pltpu.VMEM((CR,), jnp.int32),
          pltpu.VMEM((BPW,), jnp.int32),
          pltpu.VMEM((CR, D), jnp.float32),
          pltpu.VMEM((BPW, D), jnp.float32),
          pltpu.VMEM((BPW, D), jnp.float32),
          pltpu.VMEM((BPW, D), jnp.float32),
          pltpu.VMEM((L,), jnp.float32),
      ] + [pltpu.SemaphoreType.DMA] * (len(range(0, CR, CHUNK)) + 1),
  )
  def body(table_hbm, idxc_hbm, idxn_hbm, x_hbm, cen_hbm,
           idxc_v, idxn_v, rows_v, cen_v, xv, cennv, red_v, *sems):
    wid = lax.axis_index("s") * NC + lax.axis_index("c")
    pltpu.sync_copy(idxc_hbm.at[pl.ds(wid * CR, CR)], idxc_v)
    pltpu.sync_copy(idxn_hbm.at[pl.ds(wid * BPW, BPW)], idxn_v)
    chunks = [(k, min(CHUNK, CR - k)) for k in range(0, CR, CHUNK)]
    copies = [
        pltpu.async_copy(table_hbm.at[idxc_v.at[pl.ds(k, sz)]],
                         rows_v.at[pl.ds(k, sz)], sems[ci])
        for ci, (k, sz) in enumerate(chunks)
    ]
    cen_copy = pltpu.async_copy(table_hbm.at[idxn_v], cen_v, sems[-1])
    cen_copy.wait()

    def _norm_row(ref, r):
      vs = [ref[r, pl.ds(k * L, L)] for k in range(NV)]
      ss = vs[0] * vs[0]
      for k in range(1, NV):
        ss = ss + vs[k] * vs[k]
      tot = ss[0]
      for l in range(1, L):
        tot = tot + ss[l]
      inv = _rsqrt_bcast(tot)
      return [v * inv for v in vs]

    def _ctx_body(b, _):
      acc = None
      for m in range(W):
        nv = _norm_row(rows_v, b * W + m)
        acc = nv if acc is None else [a + v for a, v in zip(acc, nv)]
      for k in range(NV):
        xv[b, pl.ds(k * L, L)] = acc[k]
      nc = _norm_row(cen_v, b)
      for k in range(NV):
        cennv[b, pl.ds(k * L, L)] = nc[k]
      return 0

    for c in copies:
      c.wait()
    lax.fori_loop(0, BPW, _ctx_body, 0)
    pltpu.sync_copy(xv, x_hbm.at[pl.ds(wid * BPW, BPW)])
    pltpu.sync_copy(cennv, cen_hbm.at[pl.ds(wid * BPW, BPW)])

  return body(table, idx_ctx, idx_cen)


def _fused_body(x_in, cen_ref, w_ref, loss_ref, xb_ref, acc_ref):
  i = pl.program_id(0)

  @pl.when(i == 0)
  def _init():
    xb_ref[...] = x_in[...].astype(jnp.bfloat16)
    acc_ref[...] = jnp.zeros_like(acc_ref)

  w = w_ref[...]
  row = i * TV + lax.broadcasted_iota(jnp.int32, (TV, 1), 0)
  ss = jnp.sum(w * w, axis=1, keepdims=True)
  ss = jnp.where(row < V, ss, 1e30)
  inv = lax.rsqrt(ss) * LOG2E
  wn = (w * inv).astype(jnp.bfloat16)
  xb = xb_ref[...]

  def _lanesum(es):
    p = es[:, 0:D]
    for c in range(D, es.shape[1], D):
      p = p + es[:, c:c + D]
    return p

  CH = 512
  part = None
  prev = None
  for t in range(TV // CH):
    st = lax.dot_general(xb, wn[t * CH:(t + 1) * CH],
                         (((1,), (1,)), ((), ())),
                         preferred_element_type=jnp.float32)
    if prev is not None:
      p = _lanesum(jnp.exp2(prev))
      part = p if part is None else part + p
    prev = st
  part = part + _lanesum(jnp.exp2(prev))
  acc_ref[...] += part

  @pl.when(i == NT - 1)
  def _epilogue():
    tgt = jnp.sum(x_in[...] * cen_ref[...], axis=1, keepdims=True)
    z = jnp.sum(acc_ref[...], axis=1, keepdims=True) - NPAD
    lvec = jnp.log(z) - tgt
    loss_ref[...] = jnp.sum(lvec, axis=0, keepdims=True) / B


def kernel(context_words, central_words, negative_sampling, weight):
  del negative_sampling
  idx_ctx = context_words.reshape(-1).astype(jnp.int32)
  idx_cen = central_words.reshape(-1).astype(jnp.int32)
  x, cenn = _sc_gather_normalize(weight, idx_ctx, idx_cen)

  loss = pl.pallas_call(
      _fused_body,
      grid=(NT,),
      in_specs=[
          pl.BlockSpec((B, D), lambda i: (0, 0)),
          pl.BlockSpec((B, D), lambda i: (0, 0)),
          pl.BlockSpec((TV, D), lambda i: (i, 0)),
      ],
      out_specs=pl.BlockSpec((1, 1), lambda i: (0, 0)),
      out_shape=jax.ShapeDtypeStruct((1, 1), jnp.float32),
      scratch_shapes=[
          pltpu.VMEM((B, D), jnp.bfloat16),
          pltpu.VMEM((B, D), jnp.float32),
      ],
  )(x, cenn, weight)
  return loss[0, 0]


# speedup vs baseline: 1.0119x; 1.0119x over previous
import functools

import jax
import jax.numpy as jnp
from jax import lax
from jax.experimental import pallas as pl
from jax.experimental.pallas import tpu as pltpu
from jax.experimental.pallas import tpu_sc as plsc

V = 100000
D = 128
B = 1024
W = 20

NC = 2
NS = 16
NW = NC * NS
BPW = B // NW
CR = BPW * W
CHUNK = 128
L = 16
NV = D // L

TV = 4096
NT = (V + TV - 1) // TV
LOG2E = 1.4426950408889634
NPAD = NT * TV - V


def _rsqrt_bcast(tot):
  i = lax.bitcast_convert_type(tot, jnp.int32)
  y0 = lax.bitcast_convert_type(
      jnp.int32(0x5F3759DF) - lax.shift_right_logical(i, 1), jnp.float32)
  y = jnp.full((L,), y0, jnp.float32)
  t = jnp.full((L,), tot, jnp.float32)
  for _ in range(2):
    y = y * (1.5 - 0.5 * t * y * y)
  return y


def _sc_gather_normalize(table, idx_ctx, idx_cen):
  mesh = plsc.VectorSubcoreMesh(core_axis_name="c", subcore_axis_name="s")

  @functools.partial(
      pl.kernel,
      out_type=(
          jax.ShapeDtypeStruct((B, D), jnp.float32),
          jax.ShapeDtypeStruct((B, D), jnp.float32),
      ),
      mesh=mesh,
      scratch_types=[
          pltpu.VMEM((CR,), jnp.int32),
          pltpu.VMEM((BPW,), jnp.int32),
          pltpu.VMEM((CR, D), jnp.float32),
          pltpu.VMEM((BPW, D), jnp.float32),
          pltpu.VMEM((BPW, D), jnp.float32),
          pltpu.VMEM((BPW, D), jnp.float32),
          pltpu.VMEM((L,), jnp.float32),
      ] + [pltpu.SemaphoreType.DMA] * (len(range(0, CR, CHUNK)) + 1),
  )
  def body(table_hbm, idxc_hbm, idxn_hbm, x_hbm, cen_hbm,
           idxc_v, idxn_v, rows_v, cen_v, xv, cennv, red_v, *sems):
    wid = lax.axis_index("s") * NC + lax.axis_index("c")
    pltpu.sync_copy(idxc_hbm.at[pl.ds(wid * CR, CR)], idxc_v)
    pltpu.sync_copy(idxn_hbm.at[pl.ds(wid * BPW, BPW)], idxn_v)
    chunks = [(k, min(CHUNK, CR - k)) for k in range(0, CR, CHUNK)]
    copies = [
        pltpu.async_copy(table_hbm.at[idxc_v.at[pl.ds(k, sz)]],
                         rows_v.at[pl.ds(k, sz)], sems[ci])
        for ci, (k, sz) in enumerate(chunks)
    ]
    cen_copy = pltpu.async_copy(table_hbm.at[idxn_v], cen_v, sems[-1])
    cen_copy.wait()

    def _norm_row(ref, r):
      vs = [ref[r, pl.ds(k * L, L)] for k in range(NV)]
      ss = vs[0] * vs[0]
      for k in range(1, NV):
        ss = ss + vs[k] * vs[k]
      tot = ss[0]
      for l in range(1, L):
        tot = tot + ss[l]
      inv = _rsqrt_bcast(tot)
      return [v * inv for v in vs]

    def _ctx_body(b, _):
      acc = None
      for m in range(W):
        nv = _norm_row(rows_v, b * W + m)
        acc = nv if acc is None else [a + v for a, v in zip(acc, nv)]
      for k in range(NV):
        xv[b, pl.ds(k * L, L)] = acc[k]
      nc = _norm_row(cen_v, b)
      for k in range(NV):
        cennv[b, pl.ds(k * L, L)] = nc[k]
      return 0

    for c in copies:
      c.wait()
    lax.fori_loop(0, BPW, _ctx_body, 0)
    pltpu.sync_copy(xv, x_hbm.at[pl.ds(wid * BPW, BPW)])
    pltpu.sync_copy(cennv, cen_hbm.at[pl.ds(wid * BPW, BPW)])

  return body(table, idx_ctx, idx_cen)


def _fused_body(x_in, cen_ref, w_ref, loss_ref, xb_ref, acc_ref):
  i = pl.program_id(0)

  @pl.when(i == 0)
  def _init():
    xb_ref[...] = x_in[...].astype(jnp.bfloat16)
    acc_ref[...] = jnp.zeros_like(acc_ref)

  def _lanesum(es):
    p = es[:, 0:D]
    for c in range(D, es.shape[1], D):
      p = p + es[:, c:c + D]
    return p

  def _vocab_step(masked):
    w = w_ref[...]
    ss = jnp.sum(w * w, axis=1, keepdims=True)
    if masked:
      row = i * TV + lax.broadcasted_iota(jnp.int32, (TV, 1), 0)
      ss = jnp.where(row < V, ss, 1e30)
    inv = lax.rsqrt(ss) * LOG2E
    wn = (w * inv).astype(jnp.bfloat16)
    xb = xb_ref[...]

    CH = 512
    part = None
    prev = None
    for t in range(TV // CH):
      st = lax.dot_general(xb, wn[t * CH:(t + 1) * CH],
                           (((1,), (1,)), ((), ())),
                           preferred_element_type=jnp.float32)
      if prev is not None:
        p = _lanesum(jnp.exp2(prev))
        part = p if part is None else part + p
      prev = st
    part = part + _lanesum(jnp.exp2(prev))
    acc_ref[...] += part

  pl.when(i < NT - 1)(lambda: _vocab_step(False))
  pl.when(i == NT - 1)(lambda: _vocab_step(True))

  @pl.when(i == NT - 1)
  def _epilogue():
    tgt = jnp.sum(x_in[...] * cen_ref[...], axis=1, keepdims=True)
    z = jnp.sum(acc_ref[...], axis=1, keepdims=True) - NPAD
    lvec = jnp.log(z) - tgt
    loss_ref[...] = jnp.sum(lvec, axis=0, keepdims=True) / B


def kernel(context_words, central_words, negative_sampling, weight):
  del negative_sampling
  idx_ctx = context_words.reshape(-1).astype(jnp.int32)
  idx_cen = central_words.reshape(-1).astype(jnp.int32)
  x, cenn = _sc_gather_normalize(weight, idx_ctx, idx_cen)

  loss = pl.pallas_call(
      _fused_body,
      grid=(NT,),
      in_specs=[
          pl.BlockSpec((B, D), lambda i: (0, 0)),
          pl.BlockSpec((B, D), lambda i: (0, 0)),
          pl.BlockSpec((TV, D), lambda i: (i, 0)),
      ],
      out_specs=pl.BlockSpec((1, 1), lambda i: (0, 0)),
      out_shape=jax.ShapeDtypeStruct((1, 1), jnp.float32),
      scratch_shapes=[
          pltpu.VMEM((B, D), jnp.bfloat16),
          pltpu.VMEM((B, D), jnp.float32),
      ],
  )(x, cenn, weight)
  return loss[0, 0]
